# baseline (speedup 1.0000x reference)
import functools

import jax
import jax.numpy as jnp
from jax import lax
from jax.experimental import pallas as pl
from jax.experimental.pallas import tpu as pltpu
from jax.experimental.pallas import tpu_sc as plsc

NC = 2
NS = 16
NW = NC * NS

NB = 4096
S = 26
D = 64
BPW = NB // NW
NCHUNK = BPW
NBUF = 8


def _sc_gather(table, idx):
    mesh = plsc.VectorSubcoreMesh(
        core_axis_name="c", subcore_axis_name="s",
        num_cores=NC, num_subcores=NS,
    )

    @functools.partial(
        pl.kernel,
        out_type=jax.ShapeDtypeStruct((NB, S, D), jnp.float32),
        mesh=mesh,
        scratch_types=[
            pltpu.VMEM((BPW, S), jnp.int32),
            pltpu.VMEM((NBUF, S, D), jnp.float32),
            pltpu.SemaphoreType.DMA((NBUF,)),
            pltpu.SemaphoreType.DMA((NBUF,)),
        ],
        compiler_params=pltpu.CompilerParams(use_tc_tiling_on_sc=False),
    )
    def k(table_hbm, idx_hbm, out_hbm, idx_v, rows_v, gsem, ssem):
        wid = lax.axis_index("s") * NC + lax.axis_index("c")
        base = wid * BPW
        pltpu.sync_copy(idx_hbm.at[pl.ds(base, BPW)], idx_v)

        def fire_gather(j, b):
            pltpu.async_copy(table_hbm.at[idx_v.at[j]],
                             rows_v.at[b], gsem.at[b])

        for b in range(NBUF):
            fire_gather(b, b)

        @pl.loop(0, NCHUNK, step=NBUF)
        def _(j0):
            for b in range(NBUF):
                j = j0 + b
                pltpu.make_async_copy(
                    table_hbm.at[idx_v.at[j]],
                    rows_v.at[b], gsem.at[b]).wait()
                pltpu.async_copy(
                    rows_v.at[b], out_hbm.at[base + j], ssem.at[b])
                pltpu.make_async_copy(
                    rows_v.at[b], out_hbm.at[base + j], ssem.at[b]).wait()

                nxt = j + NBUF

                @pl.when(nxt < NCHUNK)
                def _():
                    fire_gather(nxt, b)

    return k(table, idx)


@jax.jit
def kernel(ents, table):
    return _sc_gather(table, ents.astype(jnp.int32))


# speedup vs baseline: 1.0084x; 1.0084x over previous
import functools

import jax
import jax.numpy as jnp
from jax import lax
from jax.experimental import pallas as pl
from jax.experimental.pallas import tpu as pltpu
from jax.experimental.pallas import tpu_sc as plsc

NC = 2
NS = 16
NW = NC * NS

B = 4096 * 26
D = 64
CHUNK = 256
PER_W = B // NW
NCHUNK = PER_W // CHUNK
NBUF = 4


def _sc_gather(table, idx):
    mesh = plsc.VectorSubcoreMesh(
        core_axis_name="c", subcore_axis_name="s",
        num_cores=NC, num_subcores=NS,
    )

    @functools.partial(
        pl.kernel,
        out_type=jax.ShapeDtypeStruct((B, D), jnp.float32),
        mesh=mesh,
        scratch_types=[
            pltpu.VMEM((NCHUNK, CHUNK), jnp.int32),
            pltpu.VMEM((NBUF, CHUNK, D), jnp.float32),
            pltpu.SemaphoreType.DMA((NBUF,)),
            pltpu.SemaphoreType.DMA((NBUF,)),
        ],
        compiler_params=pltpu.CompilerParams(use_tc_tiling_on_sc=False),
    )
    def k(table_hbm, idx_hbm, out_hbm, idx_v, rows_v, gsem, ssem):
        wid = lax.axis_index("s") * NC + lax.axis_index("c")
        pltpu.sync_copy(idx_hbm.at[wid], idx_v)
        base = wid * PER_W

        def fire_gather(j, b):
            pltpu.async_copy(table_hbm.at[idx_v.at[j]], rows_v.at[b],
                             gsem.at[b])

        for b in range(min(NBUF, NCHUNK)):
            fire_gather(b, b)

        @pl.loop(0, NCHUNK + NBUF - 1 - ((NCHUNK - 1) % NBUF), step=NBUF)
        def _(j0):
            for b in range(NBUF):
                j = j0 + b

                @pl.when(j < NCHUNK)
                def _():
                    pltpu.make_async_copy(table_hbm.at[idx_v.at[j]],
                                          rows_v.at[b], gsem.at[b]).wait()
                    pltpu.async_copy(
                        rows_v.at[b],
                        out_hbm.at[pl.ds(base + j * CHUNK, CHUNK)],
                        ssem.at[b])
                    pltpu.make_async_copy(
                        rows_v.at[b],
                        out_hbm.at[pl.ds(base + j * CHUNK, CHUNK)],
                        ssem.at[b]).wait()

                    nxt = j + NBUF

                    @pl.when(nxt < NCHUNK)
                    def _():
                        fire_gather(nxt, b)

    return k(table, idx)


@jax.jit
def kernel(ents, table):
    idx = ents.astype(jnp.int32).reshape(NW, NCHUNK, CHUNK)
    out = _sc_gather(table, idx)
    return out.reshape(ents.shape[0], ents.shape[1], D)


# speedup vs baseline: 1.2669x; 1.2563x over previous
import functools

import jax
import jax.numpy as jnp
from jax import lax
from jax.experimental import pallas as pl
from jax.experimental.pallas import tpu as pltpu
from jax.experimental.pallas import tpu_sc as plsc

NC = 2
NS = 16
NW = NC * NS

B = 4096 * 26
D = 64
CHUNK = 256
PER_W = B // NW
NCHUNK = PER_W // CHUNK
NBUF = 4


def _sc_gather(table, idx):
    mesh = plsc.VectorSubcoreMesh(
        core_axis_name="c", subcore_axis_name="s",
        num_cores=NC, num_subcores=NS,
    )

    @functools.partial(
        pl.kernel,
        out_type=jax.ShapeDtypeStruct((B, D), jnp.float32),
        mesh=mesh,
        scratch_types=[
            pltpu.VMEM((NCHUNK, CHUNK), jnp.int32),
            pltpu.VMEM((NBUF, CHUNK, D), jnp.float32),
            pltpu.SemaphoreType.DMA((NBUF,)),
            pltpu.SemaphoreType.DMA((NBUF,)),
        ],
        compiler_params=pltpu.CompilerParams(use_tc_tiling_on_sc=False),
    )
    def k(table_hbm, idx_hbm, out_hbm, idx_v, rows_v, gsem, ssem):
        wid = lax.axis_index("s") * NC + lax.axis_index("c")
        pltpu.sync_copy(idx_hbm.at[wid], idx_v)
        base = wid * PER_W

        def fire_gather(j, b):
            pltpu.async_copy(table_hbm.at[idx_v.at[j]], rows_v.at[b],
                             gsem.at[b])

        for b in range(min(NBUF, NCHUNK)):
            fire_gather(b, b)

        @pl.loop(0, NCHUNK + NBUF - 1 - ((NCHUNK - 1) % NBUF), step=NBUF)
        def _(j0):
            for b in range(NBUF):
                j = j0 + b

                @pl.when(j < NCHUNK)
                def _():
                    pltpu.make_async_copy(table_hbm.at[idx_v.at[j]],
                                          rows_v.at[b], gsem.at[b]).wait()
                    pltpu.async_copy(
                        rows_v.at[b],
                        out_hbm.at[pl.ds(base + j * CHUNK, CHUNK)],
                        ssem.at[b])
                    pltpu.make_async_copy(
                        rows_v.at[b],
                        out_hbm.at[pl.ds(base + j * CHUNK, CHUNK)],
                        ssem.at[b]).wait()

                    nxt = j + NBUF

                    @pl.when(nxt < NCHUNK)
                    def _():
                        fire_gather(nxt, b)

    return k(table, idx)


BB = 128


def _tc_c_body(x_ref, o_ref):
    x = x_ref[...]
    y = x.reshape(BB, 26 * D)
    o_ref[...] = y.T


def _tc_out_relayout(flat):
    return pl.pallas_call(
        _tc_c_body,
        grid=(4096 // BB,),
        in_specs=[pl.BlockSpec((13 * BB, 128), lambda i: (i, 0))],
        out_specs=pl.BlockSpec((26 * D, BB), lambda i: (0, i)),
        out_shape=jax.ShapeDtypeStruct((26 * D, 4096), jnp.float32),
    )(flat)


@jax.jit
def kernel(ents, table):
    idx = ents.astype(jnp.int32).reshape(NW, NCHUNK, CHUNK)
    out = _sc_gather(table, idx)
    res = _tc_out_relayout(out.reshape(53248, 128))
    return res.reshape(26, D, 4096).transpose(2, 0, 1)
